# baseline (speedup 1.0000x reference)
import functools

import jax
import jax.numpy as jnp
from jax import lax
from jax.experimental import pallas as pl
from jax.experimental.pallas import tpu as pltpu

N = 10000
IN = 256
OUT = 256
H = 4
HALF = OUT // 2

_NB = 10
_BR = N // _NB


def _dense_body(x_ref, w_ref, asrc_ref, adst_ref, xp_ref, a8_ref):
    xb = x_ref[...]
    xp = jnp.dot(xb, w_ref[...], preferred_element_type=jnp.float32)
    lo = [xp[:, h * OUT : h * OUT + HALF] for h in range(H)]
    hi = [xp[:, h * OUT + HALF : (h + 1) * OUT] for h in range(H)]
    xp_ref[0] = jnp.concatenate(lo, axis=1)
    xp_ref[1] = jnp.concatenate(hi, axis=1)
    for h in range(H):
        blk = xp[:, h * OUT : (h + 1) * OUT]
        a8_ref[:, h : h + 1] = jnp.sum(
            blk * asrc_ref[h, :][None, :], axis=1, keepdims=True
        )
        a8_ref[:, H + h : H + h + 1] = jnp.sum(
            blk * adst_ref[h, :][None, :], axis=1, keepdims=True
        )


def _dense_phase(x, W, att_src, att_dst):
    return pl.pallas_call(
        _dense_body,
        grid=(_NB,),
        in_specs=[
            pl.BlockSpec((_BR, IN), lambda i: (i, 0)),
            pl.BlockSpec((IN, H * OUT), lambda i: (0, 0)),
            pl.BlockSpec((H, OUT), lambda i: (0, 0)),
            pl.BlockSpec((H, OUT), lambda i: (0, 0)),
        ],
        out_specs=[
            pl.BlockSpec((2, _BR, H * HALF), lambda i: (0, i, 0)),
            pl.BlockSpec((_BR, 2 * H), lambda i: (i, 0)),
        ],
        out_shape=[
            jax.ShapeDtypeStruct((2, N, H * HALF), jnp.float32),
            jax.ShapeDtypeStruct((N, 2 * H), jnp.float32),
        ],
    )(x, W, att_src, att_dst)


def _combine_body(acc_ref, x_ref, bias_ref, g1w_ref, g1b_ref, g2w_ref, g2b_ref, o_ref):
    xb = x_ref[...]
    s1 = jax.nn.sigmoid(
        jnp.dot(xb, g1w_ref[...], preferred_element_type=jnp.float32)
        + g1b_ref[...][None, :]
    )
    gate = jax.nn.sigmoid(
        jnp.dot(s1, g2w_ref[...], preferred_element_type=jnp.float32)
        + g2b_ref[...][None, :]
    )
    attn = jnp.concatenate([acc_ref[0], acc_ref[1]], axis=1) + bias_ref[...][None, :]
    o_ref[...] = attn * gate + xb


def _combine_phase(acc, x, bias, g1_w, g1_b, g2_w, g2_b):
    return pl.pallas_call(
        _combine_body,
        grid=(_NB,),
        in_specs=[
            pl.BlockSpec((2, _BR, HALF), lambda i: (0, i, 0)),
            pl.BlockSpec((_BR, IN), lambda i: (i, 0)),
            pl.BlockSpec((OUT,), lambda i: (0,)),
            pl.BlockSpec((IN, 8), lambda i: (0, 0)),
            pl.BlockSpec((8,), lambda i: (0,)),
            pl.BlockSpec((8, OUT), lambda i: (0, 0)),
            pl.BlockSpec((OUT,), lambda i: (0,)),
        ],
        out_specs=pl.BlockSpec((_BR, OUT), lambda i: (i, 0)),
        out_shape=jax.ShapeDtypeStruct((N, OUT), jnp.float32),
    )(acc, x, bias, g1_w, g1_b, g2_w, g2_b)


def kernel(x, edge_index, W, att_src, att_dst, bias, g1_w, g1_b, g2_w, g2_b):
    loops = jnp.arange(N, dtype=edge_index.dtype)
    ei = jnp.concatenate([edge_index, jnp.stack([loops, loops], axis=0)], axis=1)
    src, dst = ei[0], ei[1]

    xp_hc, a8 = _dense_phase(x, W, att_src, att_dst)

    alpha = a8[src, :H] + a8[dst, H:]
    alpha = jnp.where(alpha > 0, alpha, 0.2 * alpha)
    ex = jnp.exp(alpha)
    denom = jax.ops.segment_sum(ex, dst, num_segments=N)
    w = ex / (denom[dst] + 1e-16) * 0.25
    xph = xp_hc.reshape(2, N, H, HALF)
    acc0 = jax.ops.segment_sum(
        jnp.sum(xph[0][src] * w[:, :, None], axis=1), dst, num_segments=N
    )
    acc1 = jax.ops.segment_sum(
        jnp.sum(xph[1][src] * w[:, :, None], axis=1), dst, num_segments=N
    )
    acc = jnp.stack([acc0, acc1], axis=0)

    return _combine_phase(acc, x, bias, g1_w, g1_b, g2_w, g2_b)
